# Initial kernel scaffold; baseline (speedup 1.0000x reference)
import jax
import jax.numpy as jnp
from jax.experimental import pallas as pl


def kernel(x, W, b):
    raise NotImplementedError("write your pallas kernel here")


# speedup vs baseline: 2.3396x; 2.3396x over previous
import functools

import jax
import jax.numpy as jnp
from jax.experimental import pallas as pl
from jax.experimental.pallas import tpu as pltpu

_INPUT = 768
_EXPERTS = 64
_BLOCK_T = 2048


def _gate_topk_kernel(x_ref, wt_ref, b_ref, ps_ref, pi_ref):
    x = x_ref[...]
    scores = jnp.dot(x, wt_ref[...], preferred_element_type=jnp.float32)
    scores = scores + b_ref[...]

    col = jax.lax.broadcasted_iota(jnp.int32, scores.shape, 1)
    neg_inf = jnp.float32(-jnp.inf)

    v1 = jnp.max(scores, axis=1, keepdims=True)
    i1 = jnp.min(jnp.where(scores == v1, col, _EXPERTS), axis=1, keepdims=True)

    scores2 = jnp.where(col == i1, neg_inf, scores)
    v2 = jnp.max(scores2, axis=1, keepdims=True)
    i2 = jnp.min(jnp.where(scores2 == v2, col, _EXPERTS), axis=1, keepdims=True)

    e2 = jnp.exp(v2 - v1)
    denom = 1.0 + e2
    p1 = 1.0 / denom
    p2 = e2 / denom

    ps_ref[...] = jnp.concatenate([p1, p2], axis=1)
    pi_ref[...] = jnp.concatenate([i1, i2], axis=1)


@jax.jit
def kernel(x, W, b):
    n_tokens = x.shape[0]
    wt = W.T
    b2 = b.reshape(1, _EXPERTS)
    grid = (n_tokens // _BLOCK_T,)
    ps, pi = pl.pallas_call(
        _gate_topk_kernel,
        grid=grid,
        in_specs=[
            pl.BlockSpec((_BLOCK_T, _INPUT), lambda i: (i, 0)),
            pl.BlockSpec((_INPUT, _EXPERTS), lambda i: (0, 0)),
            pl.BlockSpec((1, _EXPERTS), lambda i: (0, 0)),
        ],
        out_specs=[
            pl.BlockSpec((_BLOCK_T, 2), lambda i: (i, 0)),
            pl.BlockSpec((_BLOCK_T, 2), lambda i: (i, 0)),
        ],
        out_shape=[
            jax.ShapeDtypeStruct((n_tokens, 2), jnp.float32),
            jax.ShapeDtypeStruct((n_tokens, 2), jnp.int32),
        ],
    )(x, wt, b2)
    return ps, pi


# speedup vs baseline: 2.5151x; 1.0750x over previous
import functools

import jax
import jax.numpy as jnp
from jax.experimental import pallas as pl
from jax.experimental.pallas import tpu as pltpu

_INPUT = 768
_EXPERTS = 64
_BLOCK_T = 4096


def _gate_topk_kernel(x_ref, wt_ref, b_ref, ps_ref, pi_ref):
    x = x_ref[...]
    scores = jnp.dot(x, wt_ref[...], preferred_element_type=jnp.float32)
    scores = scores + b_ref[...]

    col = jax.lax.broadcasted_iota(jnp.int32, scores.shape, 1)
    neg_inf = jnp.float32(-jnp.inf)

    v1 = jnp.max(scores, axis=1, keepdims=True)
    i1 = jnp.min(jnp.where(scores == v1, col, _EXPERTS), axis=1, keepdims=True)

    scores2 = jnp.where(col == i1, neg_inf, scores)
    v2 = jnp.max(scores2, axis=1, keepdims=True)
    i2 = jnp.min(jnp.where(scores2 == v2, col, _EXPERTS), axis=1, keepdims=True)

    e2 = jnp.exp(v2 - v1)
    denom = 1.0 + e2
    p1 = 1.0 / denom
    p2 = e2 / denom

    ps_ref[...] = jnp.concatenate([p1, p2], axis=1)
    pi_ref[...] = jnp.concatenate([i1, i2], axis=1)


@jax.jit
def kernel(x, W, b):
    n_tokens = x.shape[0]
    wt = W.T
    b2 = b.reshape(1, _EXPERTS)
    grid = (n_tokens // _BLOCK_T,)
    ps, pi = pl.pallas_call(
        _gate_topk_kernel,
        grid=grid,
        in_specs=[
            pl.BlockSpec((_BLOCK_T, _INPUT), lambda i: (i, 0)),
            pl.BlockSpec((_INPUT, _EXPERTS), lambda i: (0, 0)),
            pl.BlockSpec((1, _EXPERTS), lambda i: (0, 0)),
        ],
        out_specs=[
            pl.BlockSpec((_BLOCK_T, 2), lambda i: (i, 0)),
            pl.BlockSpec((_BLOCK_T, 2), lambda i: (i, 0)),
        ],
        out_shape=[
            jax.ShapeDtypeStruct((n_tokens, 2), jnp.float32),
            jax.ShapeDtypeStruct((n_tokens, 2), jnp.int32),
        ],
    )(x, wt, b2)
    return ps, pi


# speedup vs baseline: 2.5830x; 1.0270x over previous
import functools

import jax
import jax.numpy as jnp
from jax.experimental import pallas as pl
from jax.experimental.pallas import tpu as pltpu

_INPUT = 768
_EXPERTS = 64
_BLOCK_T = 4096


def _gate_topk_kernel(x_ref, w_ref, b_ref, ps_ref, pi_ref):
    x = x_ref[...]
    scores = jax.lax.dot_general(
        x, w_ref[...], (((1,), (1,)), ((), ())),
        preferred_element_type=jnp.float32,
    )
    scores = scores + b_ref[...]

    col = jax.lax.broadcasted_iota(jnp.int32, scores.shape, 1)
    neg_inf = jnp.float32(-jnp.inf)

    v1 = jnp.max(scores, axis=1, keepdims=True)
    i1 = jnp.min(jnp.where(scores == v1, col, _EXPERTS), axis=1, keepdims=True)

    scores2 = jnp.where(col == i1, neg_inf, scores)
    v2 = jnp.max(scores2, axis=1, keepdims=True)
    i2 = jnp.min(jnp.where(scores2 == v2, col, _EXPERTS), axis=1, keepdims=True)

    e2 = jnp.exp(v2 - v1)
    denom = 1.0 + e2
    p1 = 1.0 / denom
    p2 = e2 / denom

    ps_ref[...] = jnp.concatenate([p1, p2], axis=1)
    pi_ref[...] = jnp.concatenate([i1, i2], axis=1)


@jax.jit
def kernel(x, W, b):
    n_tokens = x.shape[0]
    b2 = b.reshape(1, _EXPERTS)
    grid = (n_tokens // _BLOCK_T,)
    ps, pi = pl.pallas_call(
        _gate_topk_kernel,
        grid=grid,
        in_specs=[
            pl.BlockSpec((_BLOCK_T, _INPUT), lambda i: (i, 0)),
            pl.BlockSpec((_EXPERTS, _INPUT), lambda i: (0, 0)),
            pl.BlockSpec((1, _EXPERTS), lambda i: (0, 0)),
        ],
        out_specs=[
            pl.BlockSpec((_BLOCK_T, 2), lambda i: (i, 0)),
            pl.BlockSpec((_BLOCK_T, 2), lambda i: (i, 0)),
        ],
        out_shape=[
            jax.ShapeDtypeStruct((n_tokens, 2), jnp.float32),
            jax.ShapeDtypeStruct((n_tokens, 2), jnp.int32),
        ],
    )(x, W, b2)
    return ps, pi


# speedup vs baseline: 2.5861x; 1.0012x over previous
import functools

import jax
import jax.numpy as jnp
from jax.experimental import pallas as pl
from jax.experimental.layout import Format, Layout
from jax.experimental.pallas import tpu as pltpu

_INPUT = 768
_EXPERTS = 64
_BLOCK_T = 4096


def _gate_topk_kernel(x_ref, w_ref, b_ref, ps_ref, pi_ref):
    x = x_ref[...]
    scores = jax.lax.dot_general(
        x, w_ref[...], (((1,), (1,)), ((), ())),
        preferred_element_type=jnp.float32,
    )
    scores = scores + b_ref[...]

    col = jax.lax.broadcasted_iota(jnp.int32, scores.shape, 1)
    neg_inf = jnp.float32(-jnp.inf)

    v1 = jnp.max(scores, axis=1, keepdims=True)
    i1 = jnp.min(jnp.where(scores == v1, col, _EXPERTS), axis=1, keepdims=True)

    scores2 = jnp.where(col == i1, neg_inf, scores)
    v2 = jnp.max(scores2, axis=1, keepdims=True)
    i2 = jnp.min(jnp.where(scores2 == v2, col, _EXPERTS), axis=1, keepdims=True)

    e2 = jnp.exp(v2 - v1)
    denom = 1.0 + e2
    p1 = 1.0 / denom
    p2 = e2 / denom

    ps_ref[...] = jnp.concatenate([p1, p2], axis=1)
    pi_ref[...] = jnp.concatenate([i1, i2], axis=1)


def _impl(x, W, b):
    n_tokens = x.shape[0]
    b2 = b.reshape(1, _EXPERTS)
    grid = (n_tokens // _BLOCK_T,)
    ps, pi = pl.pallas_call(
        _gate_topk_kernel,
        grid=grid,
        in_specs=[
            pl.BlockSpec((_BLOCK_T, _INPUT), lambda i: (i, 0)),
            pl.BlockSpec((_EXPERTS, _INPUT), lambda i: (0, 0)),
            pl.BlockSpec((1, _EXPERTS), lambda i: (0, 0)),
        ],
        out_specs=[
            pl.BlockSpec((_BLOCK_T, 2), lambda i: (i, 0)),
            pl.BlockSpec((_BLOCK_T, 2), lambda i: (i, 0)),
        ],
        out_shape=[
            jax.ShapeDtypeStruct((n_tokens, 2), jnp.float32),
            jax.ShapeDtypeStruct((n_tokens, 2), jnp.int32),
        ],
    )(x, W, b2)
    return ps, pi


@functools.lru_cache(maxsize=8)
def _jitted_for_device(dev):
    fmt = Format(Layout((1, 0)), jax.sharding.SingleDeviceSharding(dev))
    return jax.jit(_impl, out_shardings=(fmt, fmt))


def kernel(x, W, b):
    try:
        dev = next(iter(x.devices()))
        return _jitted_for_device(dev)(x, W, b)
    except (AttributeError, TypeError):
        return jax.jit(_impl)(x, W, b)
